# baseline (speedup 1.0000x reference)
import functools

import jax
import jax.numpy as jnp
from jax import lax
from jax.experimental import pallas as pl
from jax.experimental.pallas import tpu as pltpu
from jax.experimental.pallas import tpu_sc as plsc

_EMBED = 64
_NC = 2
_NS = 16
_NW = _NC * _NS
_W = 128


@functools.lru_cache(maxsize=None)
def _make_sc_gather(B):
    G = B // _W
    g_per_w = G // _NW
    mesh = plsc.VectorSubcoreMesh(core_axis_name="c", subcore_axis_name="s")

    @functools.partial(
        pl.kernel,
        out_type=jax.ShapeDtypeStruct((B, _EMBED), jnp.float32),
        mesh=mesh,
        scratch_types=[
            pltpu.VMEM((g_per_w, _W), jnp.int32),
            pltpu.VMEM((_W, _EMBED), jnp.float32),
            pltpu.SemaphoreType.DMA,
        ],
        compiler_params=pltpu.CompilerParams(use_tc_tiling_on_sc=False),
    )
    def k(table_hbm, idx_hbm, out_hbm, idx_v, rows_v, sem):
        wid = lax.axis_index("s") * _NC + lax.axis_index("c")
        gbase = wid * g_per_w
        pltpu.sync_copy(idx_hbm.at[pl.ds(gbase, g_per_w)], idx_v)

        def body(j, carry):
            pltpu.async_copy(table_hbm.at[idx_v.at[j]], rows_v, sem).wait()
            pltpu.sync_copy(rows_v, out_hbm.at[pl.ds((gbase + j) * _W, _W)])
            return carry

        lax.fori_loop(0, g_per_w, body, 0)

    return k


def kernel(idx, weight):
    bsz, fields = idx.shape
    B = bsz * fields
    flat = jnp.asarray(idx, jnp.int32).reshape(B // _W, _W)
    out = _make_sc_gather(B)(weight, flat)
    return out.reshape(bsz, fields, _EMBED)


# speedup vs baseline: 1.0738x; 1.0738x over previous
import functools

import jax
import jax.numpy as jnp
from jax import lax
from jax.experimental import pallas as pl
from jax.experimental.pallas import tpu as pltpu
from jax.experimental.pallas import tpu_sc as plsc

_EMBED = 64
_NC = 2
_NS = 16
_NW = _NC * _NS
_W = 128
_K = 4


@functools.lru_cache(maxsize=None)
def _make_sc_gather(B):
    G = B // _W
    g_per_w = G // _NW
    SG = g_per_w // _K
    assert g_per_w % _K == 0 and SG % 2 == 0 and SG >= 4
    mesh = plsc.VectorSubcoreMesh(core_axis_name="c", subcore_axis_name="s")

    @functools.partial(
        pl.kernel,
        out_type=jax.ShapeDtypeStruct((B, _EMBED), jnp.float32),
        mesh=mesh,
        scratch_types=[
            pltpu.VMEM((g_per_w, _W), jnp.int32),
            pltpu.VMEM((2, _K, _W, _EMBED), jnp.float32),
            pltpu.SemaphoreType.DMA,
            pltpu.SemaphoreType.DMA,
            pltpu.SemaphoreType.DMA,
            pltpu.SemaphoreType.DMA,
        ],
        compiler_params=pltpu.CompilerParams(use_tc_tiling_on_sc=False),
    )
    def k(table_hbm, idx_hbm, out_hbm, idx_v, rows_v, g0, g1, w0, w1):
        wid = lax.axis_index("s") * _NC + lax.axis_index("c")
        gbase = wid * g_per_w
        pltpu.sync_copy(idx_hbm.at[pl.ds(gbase, g_per_w)], idx_v)
        gsem = (g0, g1)
        wsem = (w0, w1)

        def gather_desc(s, bank, b):
            return pltpu.make_async_copy(
                table_hbm.at[idx_v.at[s * _K + b]], rows_v.at[bank, b],
                gsem[bank])

        def write_desc(s, bank, b):
            return pltpu.make_async_copy(
                rows_v.at[bank, b],
                out_hbm.at[pl.ds((gbase + s * _K + b) * _W, _W)], wsem[bank])

        def fire_gathers(s, bank):
            for b in range(_K):
                gather_desc(s, bank, b).start()

        def drain_gathers(s, bank):
            for b in range(_K):
                gather_desc(s, bank, b).wait()

        def fire_writes(s, bank):
            for b in range(_K):
                write_desc(s, bank, b).start()

        def drain_writes(s, bank):
            for b in range(_K):
                write_desc(s, bank, b).wait()

        def step(s, bank):
            drain_gathers(s, bank)
            drain_writes(s - 1, 1 - bank)
            fire_gathers(s + 1, 1 - bank)
            fire_writes(s, bank)

        fire_gathers(0, 0)
        drain_gathers(0, 0)
        fire_gathers(1, 1)
        fire_writes(0, 0)

        def body(t, carry):
            step(2 * t + 1, 1)
            step(2 * t + 2, 0)
            return carry

        lax.fori_loop(0, (SG - 2) // 2, body, 0)

        drain_gathers(SG - 1, 1)
        drain_writes(SG - 2, 0)
        fire_writes(SG - 1, 1)
        drain_writes(SG - 1, 1)

    return k


def kernel(idx, weight):
    bsz, fields = idx.shape
    B = bsz * fields
    flat = jnp.asarray(idx, jnp.int32).reshape(B // _W, _W)
    out = _make_sc_gather(B)(weight, flat)
    return out.reshape(bsz, fields, _EMBED)
